# baseline (speedup 1.0000x reference)
import jax
import jax.numpy as jnp
from jax import lax
from jax.experimental import pallas as pl
from jax.experimental.pallas import tpu as pltpu
from jax.experimental.pallas import tpu_sc as plsc

N = 10000
E = 160000
D = 256
OUT = 128
HALF = 128
N_ITERS = 3

NC = 2
NS = 16
L = 16

EPT = E // NS
CHUNK = 80
NCHUNKS = EPT // CHUNK
NPAD = 10240
RPT = NPAD // NS

BN = 1000
GRID = N // BN


NSLOT = 4
NIDX = 8
IDX_LEAD = 6
NGROUPS = (NCHUNKS + NIDX - 1) // NIDX


def _edge_body(a0, a1, b0, b1, src, dst, zeros, m0, m1, *scratch):
    sidxs = scratch[:NIDX]
    didxs = scratch[NIDX:2 * NIDX]
    isems = scratch[2 * NIDX:3 * NIDX]
    bufs = scratch[3 * NIDX:3 * NIDX + NSLOT]
    sems = scratch[3 * NIDX + NSLOT:3 * NIDX + 2 * NSLOT]
    acc = scratch[3 * NIDX + 2 * NSLOT]
    c = lax.axis_index("c")
    s = lax.axis_index("s")
    r0 = s * RPT
    ebase = s * EPT

    def wait_idx(bi):
        pltpu.make_async_copy(src.at[pl.ds(0, CHUNK)], sidxs[bi],
                              isems[bi]).wait()
        pltpu.make_async_copy(src.at[pl.ds(0, CHUNK)], didxs[bi],
                              isems[bi]).wait()

    def wait_buf(b):
        pltpu.make_async_copy(a0.at[pl.ds(0, CHUNK)], bufs[b], sems[b]).wait()

    def stage_i(jx, bi):
        off = ebase + jx * CHUNK
        pltpu.async_copy(src.at[pl.ds(off, CHUNK)], sidxs[bi], isems[bi])
        pltpu.async_copy(dst.at[pl.ds(off, CHUNK)], didxs[bi], isems[bi])

    def stage_ga(jx, b, bi, reuse=True):
        if reuse:
            @pl.when(jx >= NSLOT)
            def _drain():
                wait_buf(b)
        wait_idx(bi)

        @pl.when(c == 0)
        def _g0():
            pltpu.async_copy(a0.at[sidxs[bi]], bufs[b], sems[b])

        @pl.when(c == 1)
        def _g1():
            pltpu.async_copy(a1.at[sidxs[bi]], bufs[b], sems[b])

    def stage_gb(jx, b, bi):
        wait_buf(b)

        @pl.when(c == 0)
        def _g0():
            pltpu.async_copy(b0.at[didxs[bi]], bufs[b], sems[b], add=True)

        @pl.when(c == 1)
        def _g1():
            pltpu.async_copy(b1.at[didxs[bi]], bufs[b], sems[b], add=True)

    def stage_cs(jx, b, bi):
        wait_buf(b)
        bf = bufs[b]

        def rows(rg, carry):
            for rr in range(8):
                r = rg * 8 + rr
                for q in range(HALF // L):
                    sl = pl.ds(q * L, L)
                    bf[r, sl] = jnp.maximum(bf[r, sl], 0.0)
            return carry

        lax.fori_loop(0, CHUNK // 8, rows, 0)
        pltpu.async_copy(bf, acc.at[didxs[bi]], sems[b], add=True)

    for jp in range(IDX_LEAD):
        stage_i(jp, jp % NIDX)
    stage_ga(0, 0, 0, reuse=False)
    stage_ga(1, 1, 1, reuse=False)
    stage_gb(0, 0, 0)

    pltpu.sync_copy(zeros.at[pl.ds(r0, RPT)], acc.at[pl.ds(r0, RPT)])
    plsc.subcore_barrier()

    def group(g, carry):
        for b8 in range(NIDX):
            j = g * NIDX + b8

            @pl.when(j < NCHUNKS)
            def _cs():
                stage_cs(j, b8 % NSLOT, b8)

            @pl.when(j + 1 < NCHUNKS)
            def _gb():
                stage_gb(j + 1, (b8 + 1) % NSLOT, (b8 + 1) % NIDX)

            @pl.when(j + 2 < NCHUNKS)
            def _ga():
                stage_ga(j + 2, (b8 + 2) % NSLOT, (b8 + 2) % NIDX)

            @pl.when(j + IDX_LEAD < NCHUNKS)
            def _i():
                stage_i(j + IDX_LEAD, (b8 + IDX_LEAD) % NIDX)

        return carry

    lax.fori_loop(0, NGROUPS, group, 0)

    for b in range(NSLOT):
        wait_buf(b)
    plsc.subcore_barrier()

    @pl.when(c == 0)
    def _copy0():
        pltpu.sync_copy(acc.at[pl.ds(r0, RPT)], m0.at[pl.ds(r0, RPT)])

    @pl.when(c == 1)
    def _copy1():
        pltpu.sync_copy(acc.at[pl.ds(r0, RPT)], m1.at[pl.ds(r0, RPT)])


_sc_mesh = plsc.VectorSubcoreMesh(
    core_axis_name="c", subcore_axis_name="s", num_cores=NC, num_subcores=NS)

_edge_pass = pl.kernel(
    _edge_body,
    out_type=(jax.ShapeDtypeStruct((NPAD, HALF), jnp.float32),
              jax.ShapeDtypeStruct((NPAD, HALF), jnp.float32)),
    mesh=_sc_mesh,
    scratch_types=(
        [pltpu.VMEM((CHUNK,), jnp.int32)] * (2 * NIDX)
        + [pltpu.SemaphoreType.DMA] * NIDX
        + [pltpu.VMEM((CHUNK, HALF), jnp.float32)] * NSLOT
        + [pltpu.SemaphoreType.DMA] * NSLOT
        + [pltpu.VMEM_SHARED((NPAD, HALF), jnp.float32)]
    ),
)


def _project_body(h_ref, wab_ref, bab_ref, a0_ref, a1_ref, b0_ref, b1_ref):
    p = jnp.dot(h_ref[...], wab_ref[...],
                preferred_element_type=jnp.float32) + bab_ref[...]
    a0_ref[...] = p[:, :HALF]
    a1_ref[...] = p[:, HALF:D]
    b0_ref[...] = p[:, D:D + HALF]
    b1_ref[...] = p[:, D + HALF:]


def _project(h, wab, bab):
    half_spec = pl.BlockSpec((BN, HALF), lambda i: (i, 0))
    half_shape = jax.ShapeDtypeStruct((N, HALF), jnp.float32)
    return pl.pallas_call(
        _project_body,
        grid=(GRID,),
        in_specs=[pl.BlockSpec((BN, D), lambda i: (i, 0)),
                  pl.BlockSpec((D, 2 * D), lambda i: (0, 0)),
                  pl.BlockSpec((1, 2 * D), lambda i: (0, 0))],
        out_specs=[half_spec, half_spec, half_spec, half_spec],
        out_shape=[half_shape, half_shape, half_shape, half_shape],
    )(h, wab, bab)


def _pre_body(h_ref, uzr_ref, g2_ref):
    g2_ref[...] = jnp.dot(h_ref[...], uzr_ref[...],
                          preferred_element_type=jnp.float32)


def _pre(h, uzr):
    return pl.pallas_call(
        _pre_body,
        grid=(GRID,),
        in_specs=[pl.BlockSpec((BN, D), lambda i: (i, 0)),
                  pl.BlockSpec((D, 2 * D), lambda i: (0, 0))],
        out_specs=pl.BlockSpec((BN, 2 * D), lambda i: (i, 0)),
        out_shape=jax.ShapeDtypeStruct((N, 2 * D), jnp.float32),
    )(h, uzr)


def _gru_body(m0_ref, m1_ref, h_ref, wzrh_ref, g2_ref, uh_ref, bzrh_ref,
              hn_ref):
    mb = jnp.concatenate([m0_ref[...], m1_ref[...]], axis=1)
    h = h_ref[...]
    g1 = jnp.dot(mb, wzrh_ref[...],
                 preferred_element_type=jnp.float32) + bzrh_ref[...]
    g2 = g2_ref[...]
    z = jax.nn.sigmoid(g1[:, :D] + g2[:, :D])
    r = jax.nn.sigmoid(g1[:, D:2 * D] + g2[:, D:2 * D])
    ht = jnp.tanh(g1[:, 2 * D:] + jnp.dot(r * h, uh_ref[...],
                                          preferred_element_type=jnp.float32))
    hn_ref[...] = h + z * (ht - h)


def _gru(m0, m1, h, wzrh, g2, uh, bzrh):
    return pl.pallas_call(
        _gru_body,
        grid=(GRID,),
        in_specs=[pl.BlockSpec((BN, HALF), lambda i: (i, 0)),
                  pl.BlockSpec((BN, HALF), lambda i: (i, 0)),
                  pl.BlockSpec((BN, D), lambda i: (i, 0)),
                  pl.BlockSpec((D, 3 * D), lambda i: (0, 0)),
                  pl.BlockSpec((BN, 2 * D), lambda i: (i, 0)),
                  pl.BlockSpec((D, D), lambda i: (0, 0)),
                  pl.BlockSpec((1, 3 * D), lambda i: (0, 0))],
        out_specs=pl.BlockSpec((BN, D), lambda i: (i, 0)),
        out_shape=jax.ShapeDtypeStruct((N, D), jnp.float32),
    )(m0, m1, h, wzrh, g2, uh, bzrh)


def _gru_proj_body(m0_ref, m1_ref, h_ref, wzrh_ref, g2_ref, uh_ref,
                   bzrh_ref, wab_ref, bab_ref,
                   hn_ref, a0_ref, a1_ref, b0_ref, b1_ref):
    mb = jnp.concatenate([m0_ref[...], m1_ref[...]], axis=1)
    h = h_ref[...]
    g1 = jnp.dot(mb, wzrh_ref[...],
                 preferred_element_type=jnp.float32) + bzrh_ref[...]
    g2 = g2_ref[...]
    z = jax.nn.sigmoid(g1[:, :D] + g2[:, :D])
    r = jax.nn.sigmoid(g1[:, D:2 * D] + g2[:, D:2 * D])
    ht = jnp.tanh(g1[:, 2 * D:] + jnp.dot(r * h, uh_ref[...],
                                          preferred_element_type=jnp.float32))
    hn = h + z * (ht - h)
    hn_ref[...] = hn
    p = jnp.dot(hn, wab_ref[...],
                preferred_element_type=jnp.float32) + bab_ref[...]
    a0_ref[...] = p[:, :HALF]
    a1_ref[...] = p[:, HALF:D]
    b0_ref[...] = p[:, D:D + HALF]
    b1_ref[...] = p[:, D + HALF:]


def _gru_proj(m0, m1, h, wzrh, g2, uh, bzrh, wab, bab):
    half_spec = pl.BlockSpec((BN, HALF), lambda i: (i, 0))
    half_shape = jax.ShapeDtypeStruct((N, HALF), jnp.float32)
    return pl.pallas_call(
        _gru_proj_body,
        grid=(GRID,),
        in_specs=[pl.BlockSpec((BN, HALF), lambda i: (i, 0)),
                  pl.BlockSpec((BN, HALF), lambda i: (i, 0)),
                  pl.BlockSpec((BN, D), lambda i: (i, 0)),
                  pl.BlockSpec((D, 3 * D), lambda i: (0, 0)),
                  pl.BlockSpec((BN, 2 * D), lambda i: (i, 0)),
                  pl.BlockSpec((D, D), lambda i: (0, 0)),
                  pl.BlockSpec((1, 3 * D), lambda i: (0, 0)),
                  pl.BlockSpec((D, 2 * D), lambda i: (0, 0)),
                  pl.BlockSpec((1, 2 * D), lambda i: (0, 0))],
        out_specs=[pl.BlockSpec((BN, D), lambda i: (i, 0)),
                   half_spec, half_spec, half_spec, half_spec],
        out_shape=[jax.ShapeDtypeStruct((N, D), jnp.float32),
                   half_shape, half_shape, half_shape, half_shape],
    )(m0, m1, h, wzrh, g2, uh, bzrh, wab, bab)


def _readout_body(h_ref, wg_ref, wro_ref, o_ref):
    @pl.when(pl.program_id(0) == 0)
    def _init():
        o_ref[...] = jnp.zeros_like(o_ref)

    h = h_ref[...]
    g = jax.nn.sigmoid(jnp.dot(h, wg_ref[...],
                               preferred_element_type=jnp.float32))
    ro = jnp.dot(h, wro_ref[...], preferred_element_type=jnp.float32)
    o_ref[...] += jnp.sum(g * ro, axis=0, keepdims=True)


def _readout(h, wg, wro):
    return pl.pallas_call(
        _readout_body,
        grid=(GRID,),
        in_specs=[pl.BlockSpec((BN, D), lambda i: (i, 0)),
                  pl.BlockSpec((D, OUT), lambda i: (0, 0)),
                  pl.BlockSpec((D, OUT), lambda i: (0, 0))],
        out_specs=pl.BlockSpec((1, OUT), lambda i: (0, 0)),
        out_shape=jax.ShapeDtypeStruct((1, OUT), jnp.float32),
    )(h, wg, wro)


def kernel(x, edge_index, W_msg, b_msg, W_z, U_z, b_z, W_r, U_r, b_r,
           W_h, U_h, b_h, W_g, W_ro):
    src = edge_index[0]
    dst = edge_index[1]
    wab = jnp.concatenate([W_msg[:D], W_msg[D:]], axis=1)
    bab = jnp.concatenate([b_msg, jnp.zeros((D,), jnp.float32)]
                          ).reshape(1, 2 * D)
    wzrh = jnp.concatenate([W_z, W_r, W_h], axis=1)
    uzr = jnp.concatenate([U_z, U_r], axis=1)
    bzrh = jnp.concatenate([b_z, b_r, b_h]).reshape(1, 3 * D)
    zeros_half = jnp.zeros((NPAD, HALF), jnp.float32)

    h = x
    a0, a1, b0, b1 = _project(h, wab, bab)
    for it in range(N_ITERS):
        g2 = _pre(h, uzr)
        m0, m1 = _edge_pass(a0, a1, b0, b1, src, dst, zeros_half)
        if it < N_ITERS - 1:
            h, a0, a1, b0, b1 = _gru_proj(m0, m1, h, wzrh, g2, U_h, bzrh,
                                          wab, bab)
        else:
            h = _gru(m0, m1, h, wzrh, g2, U_h, bzrh)
    return _readout(h, W_g, W_ro).reshape(OUT)


# speedup vs baseline: 1.0345x; 1.0345x over previous
import jax
import jax.numpy as jnp
from jax import lax
from jax.experimental import pallas as pl
from jax.experimental.pallas import tpu as pltpu
from jax.experimental.pallas import tpu_sc as plsc

N = 10000
E = 160000
D = 256
OUT = 128
HALF = 128
N_ITERS = 3

NC = 2
NS = 16
L = 16

EPT = E // NS
CHUNK = 80
NCHUNKS = EPT // CHUNK
NPAD = 10240
RPT = NPAD // NS

BN = 2000
GRID = N // BN


NSLOT = 4
NIDX = 8
IDX_LEAD = 6
NGROUPS = (NCHUNKS + NIDX - 1) // NIDX


def _edge_body(a0, a1, b0, b1, src, dst, zeros, m0, m1, *scratch):
    sidxs = scratch[:NIDX]
    didxs = scratch[NIDX:2 * NIDX]
    isems = scratch[2 * NIDX:3 * NIDX]
    bufs = scratch[3 * NIDX:3 * NIDX + NSLOT]
    sems = scratch[3 * NIDX + NSLOT:3 * NIDX + 2 * NSLOT]
    acc = scratch[3 * NIDX + 2 * NSLOT]
    c = lax.axis_index("c")
    s = lax.axis_index("s")
    r0 = s * RPT
    ebase = s * EPT

    def wait_idx(bi):
        pltpu.make_async_copy(src.at[pl.ds(0, CHUNK)], sidxs[bi],
                              isems[bi]).wait()
        pltpu.make_async_copy(src.at[pl.ds(0, CHUNK)], didxs[bi],
                              isems[bi]).wait()

    def wait_buf(b):
        pltpu.make_async_copy(a0.at[pl.ds(0, CHUNK)], bufs[b], sems[b]).wait()

    def stage_i(jx, bi):
        off = ebase + jx * CHUNK
        pltpu.async_copy(src.at[pl.ds(off, CHUNK)], sidxs[bi], isems[bi])
        pltpu.async_copy(dst.at[pl.ds(off, CHUNK)], didxs[bi], isems[bi])

    def stage_ga(jx, b, bi, reuse=True):
        if reuse:
            @pl.when(jx >= NSLOT)
            def _drain():
                wait_buf(b)
        wait_idx(bi)

        @pl.when(c == 0)
        def _g0():
            pltpu.async_copy(a0.at[sidxs[bi]], bufs[b], sems[b])

        @pl.when(c == 1)
        def _g1():
            pltpu.async_copy(a1.at[sidxs[bi]], bufs[b], sems[b])

    def stage_gb(jx, b, bi):
        wait_buf(b)

        @pl.when(c == 0)
        def _g0():
            pltpu.async_copy(b0.at[didxs[bi]], bufs[b], sems[b], add=True)

        @pl.when(c == 1)
        def _g1():
            pltpu.async_copy(b1.at[didxs[bi]], bufs[b], sems[b], add=True)

    def stage_cs(jx, b, bi):
        wait_buf(b)
        bf = bufs[b]

        def rows(rg, carry):
            for rr in range(8):
                r = rg * 8 + rr
                for q in range(HALF // L):
                    sl = pl.ds(q * L, L)
                    bf[r, sl] = jnp.maximum(bf[r, sl], 0.0)
            return carry

        lax.fori_loop(0, CHUNK // 8, rows, 0)
        pltpu.async_copy(bf, acc.at[didxs[bi]], sems[b], add=True)

    for jp in range(IDX_LEAD):
        stage_i(jp, jp % NIDX)
    stage_ga(0, 0, 0, reuse=False)
    stage_ga(1, 1, 1, reuse=False)
    stage_gb(0, 0, 0)

    pltpu.sync_copy(zeros.at[pl.ds(r0, RPT)], acc.at[pl.ds(r0, RPT)])
    plsc.subcore_barrier()

    def group(g, carry):
        for b8 in range(NIDX):
            j = g * NIDX + b8

            @pl.when(j < NCHUNKS)
            def _cs():
                stage_cs(j, b8 % NSLOT, b8)

            @pl.when(j + 1 < NCHUNKS)
            def _gb():
                stage_gb(j + 1, (b8 + 1) % NSLOT, (b8 + 1) % NIDX)

            @pl.when(j + 2 < NCHUNKS)
            def _ga():
                stage_ga(j + 2, (b8 + 2) % NSLOT, (b8 + 2) % NIDX)

            @pl.when(j + IDX_LEAD < NCHUNKS)
            def _i():
                stage_i(j + IDX_LEAD, (b8 + IDX_LEAD) % NIDX)

        return carry

    lax.fori_loop(0, NGROUPS, group, 0)

    for b in range(NSLOT):
        wait_buf(b)
    plsc.subcore_barrier()

    @pl.when(c == 0)
    def _copy0():
        pltpu.sync_copy(acc.at[pl.ds(r0, RPT)], m0.at[pl.ds(r0, RPT)])

    @pl.when(c == 1)
    def _copy1():
        pltpu.sync_copy(acc.at[pl.ds(r0, RPT)], m1.at[pl.ds(r0, RPT)])


_sc_mesh = plsc.VectorSubcoreMesh(
    core_axis_name="c", subcore_axis_name="s", num_cores=NC, num_subcores=NS)

_edge_pass = pl.kernel(
    _edge_body,
    out_type=(jax.ShapeDtypeStruct((NPAD, HALF), jnp.float32),
              jax.ShapeDtypeStruct((NPAD, HALF), jnp.float32)),
    mesh=_sc_mesh,
    scratch_types=(
        [pltpu.VMEM((CHUNK,), jnp.int32)] * (2 * NIDX)
        + [pltpu.SemaphoreType.DMA] * NIDX
        + [pltpu.VMEM((CHUNK, HALF), jnp.float32)] * NSLOT
        + [pltpu.SemaphoreType.DMA] * NSLOT
        + [pltpu.VMEM_SHARED((NPAD, HALF), jnp.float32)]
    ),
)


def _project_body(h_ref, wab_ref, bab_ref, a0_ref, a1_ref, b0_ref, b1_ref):
    p = jnp.dot(h_ref[...], wab_ref[...],
                preferred_element_type=jnp.float32) + bab_ref[...]
    a0_ref[...] = p[:, :HALF]
    a1_ref[...] = p[:, HALF:D]
    b0_ref[...] = p[:, D:D + HALF]
    b1_ref[...] = p[:, D + HALF:]


def _project(h, wab, bab):
    half_spec = pl.BlockSpec((BN, HALF), lambda i: (i, 0))
    half_shape = jax.ShapeDtypeStruct((N, HALF), jnp.float32)
    return pl.pallas_call(
        _project_body,
        grid=(GRID,),
        in_specs=[pl.BlockSpec((BN, D), lambda i: (i, 0)),
                  pl.BlockSpec((D, 2 * D), lambda i: (0, 0)),
                  pl.BlockSpec((1, 2 * D), lambda i: (0, 0))],
        out_specs=[half_spec, half_spec, half_spec, half_spec],
        out_shape=[half_shape, half_shape, half_shape, half_shape],
    )(h, wab, bab)


def _gru_body(m0_ref, m1_ref, h_ref, wzrh_ref, uzr_ref, uh_ref, bzrh_ref,
              hn_ref):
    mb = jnp.concatenate([m0_ref[...], m1_ref[...]], axis=1)
    h = h_ref[...]
    g1 = jnp.dot(mb, wzrh_ref[...],
                 preferred_element_type=jnp.float32) + bzrh_ref[...]
    g2 = jnp.dot(h, uzr_ref[...], preferred_element_type=jnp.float32)
    z = jax.nn.sigmoid(g1[:, :D] + g2[:, :D])
    r = jax.nn.sigmoid(g1[:, D:2 * D] + g2[:, D:2 * D])
    ht = jnp.tanh(g1[:, 2 * D:] + jnp.dot(r * h, uh_ref[...],
                                          preferred_element_type=jnp.float32))
    hn_ref[...] = h + z * (ht - h)


def _gru(m0, m1, h, wzrh, uzr, uh, bzrh):
    return pl.pallas_call(
        _gru_body,
        grid=(GRID,),
        in_specs=[pl.BlockSpec((BN, HALF), lambda i: (i, 0)),
                  pl.BlockSpec((BN, HALF), lambda i: (i, 0)),
                  pl.BlockSpec((BN, D), lambda i: (i, 0)),
                  pl.BlockSpec((D, 3 * D), lambda i: (0, 0)),
                  pl.BlockSpec((D, 2 * D), lambda i: (0, 0)),
                  pl.BlockSpec((D, D), lambda i: (0, 0)),
                  pl.BlockSpec((1, 3 * D), lambda i: (0, 0))],
        out_specs=pl.BlockSpec((BN, D), lambda i: (i, 0)),
        out_shape=jax.ShapeDtypeStruct((N, D), jnp.float32),
    )(m0, m1, h, wzrh, uzr, uh, bzrh)


def _gru_proj_body(m0_ref, m1_ref, h_ref, wzrh_ref, uzr_ref, uh_ref,
                   bzrh_ref, wab_ref, bab_ref,
                   hn_ref, a0_ref, a1_ref, b0_ref, b1_ref):
    mb = jnp.concatenate([m0_ref[...], m1_ref[...]], axis=1)
    h = h_ref[...]
    g1 = jnp.dot(mb, wzrh_ref[...],
                 preferred_element_type=jnp.float32) + bzrh_ref[...]
    g2 = jnp.dot(h, uzr_ref[...], preferred_element_type=jnp.float32)
    z = jax.nn.sigmoid(g1[:, :D] + g2[:, :D])
    r = jax.nn.sigmoid(g1[:, D:2 * D] + g2[:, D:2 * D])
    ht = jnp.tanh(g1[:, 2 * D:] + jnp.dot(r * h, uh_ref[...],
                                          preferred_element_type=jnp.float32))
    hn = h + z * (ht - h)
    hn_ref[...] = hn
    p = jnp.dot(hn, wab_ref[...],
                preferred_element_type=jnp.float32) + bab_ref[...]
    a0_ref[...] = p[:, :HALF]
    a1_ref[...] = p[:, HALF:D]
    b0_ref[...] = p[:, D:D + HALF]
    b1_ref[...] = p[:, D + HALF:]


def _gru_proj(m0, m1, h, wzrh, uzr, uh, bzrh, wab, bab):
    half_spec = pl.BlockSpec((BN, HALF), lambda i: (i, 0))
    half_shape = jax.ShapeDtypeStruct((N, HALF), jnp.float32)
    return pl.pallas_call(
        _gru_proj_body,
        grid=(GRID,),
        in_specs=[pl.BlockSpec((BN, HALF), lambda i: (i, 0)),
                  pl.BlockSpec((BN, HALF), lambda i: (i, 0)),
                  pl.BlockSpec((BN, D), lambda i: (i, 0)),
                  pl.BlockSpec((D, 3 * D), lambda i: (0, 0)),
                  pl.BlockSpec((D, 2 * D), lambda i: (0, 0)),
                  pl.BlockSpec((D, D), lambda i: (0, 0)),
                  pl.BlockSpec((1, 3 * D), lambda i: (0, 0)),
                  pl.BlockSpec((D, 2 * D), lambda i: (0, 0)),
                  pl.BlockSpec((1, 2 * D), lambda i: (0, 0))],
        out_specs=[pl.BlockSpec((BN, D), lambda i: (i, 0)),
                   half_spec, half_spec, half_spec, half_spec],
        out_shape=[jax.ShapeDtypeStruct((N, D), jnp.float32),
                   half_shape, half_shape, half_shape, half_shape],
    )(m0, m1, h, wzrh, uzr, uh, bzrh, wab, bab)


def _readout_body(h_ref, wg_ref, wro_ref, o_ref):
    @pl.when(pl.program_id(0) == 0)
    def _init():
        o_ref[...] = jnp.zeros_like(o_ref)

    h = h_ref[...]
    g = jax.nn.sigmoid(jnp.dot(h, wg_ref[...],
                               preferred_element_type=jnp.float32))
    ro = jnp.dot(h, wro_ref[...], preferred_element_type=jnp.float32)
    o_ref[...] += jnp.sum(g * ro, axis=0, keepdims=True)


def _readout(h, wg, wro):
    return pl.pallas_call(
        _readout_body,
        grid=(GRID,),
        in_specs=[pl.BlockSpec((BN, D), lambda i: (i, 0)),
                  pl.BlockSpec((D, OUT), lambda i: (0, 0)),
                  pl.BlockSpec((D, OUT), lambda i: (0, 0))],
        out_specs=pl.BlockSpec((1, OUT), lambda i: (0, 0)),
        out_shape=jax.ShapeDtypeStruct((1, OUT), jnp.float32),
    )(h, wg, wro)


def kernel(x, edge_index, W_msg, b_msg, W_z, U_z, b_z, W_r, U_r, b_r,
           W_h, U_h, b_h, W_g, W_ro):
    src = edge_index[0]
    dst = edge_index[1]
    wab = jnp.concatenate([W_msg[:D], W_msg[D:]], axis=1)
    bab = jnp.concatenate([b_msg, jnp.zeros((D,), jnp.float32)]
                          ).reshape(1, 2 * D)
    wzrh = jnp.concatenate([W_z, W_r, W_h], axis=1)
    uzr = jnp.concatenate([U_z, U_r], axis=1)
    bzrh = jnp.concatenate([b_z, b_r, b_h]).reshape(1, 3 * D)
    zeros_half = jnp.zeros((NPAD, HALF), jnp.float32)

    h = x
    a0, a1, b0, b1 = _project(h, wab, bab)
    for it in range(N_ITERS):
        m0, m1 = _edge_pass(a0, a1, b0, b1, src, dst, zeros_half)
        if it < N_ITERS - 1:
            h, a0, a1, b0, b1 = _gru_proj(m0, m1, h, wzrh, uzr, U_h, bzrh,
                                          wab, bab)
        else:
            h = _gru(m0, m1, h, wzrh, uzr, U_h, bzrh)
    return _readout(h, W_g, W_ro).reshape(OUT)
